# baseline (speedup 1.0000x reference)
import functools

import jax
import jax.numpy as jnp
from jax import lax
from jax.experimental import pallas as pl
from jax.experimental.pallas import tpu as pltpu
from jax.experimental.pallas import tpu_sc as plsc

N = 4096
D = 512
K = 2048
L = 16
NC, NS = 2, 16
NW = NC * NS
RPW_H = K // NW

T_TC = 768
SC_ROWS = K - T_TC
RPW_G = SC_ROWS // NW
GCHUNK = 5
NCHUNK = RPW_G // GCHUNK
UNROLL = 2
RB = 256


def _scores_body(h_ref, w_ref, b_ref, s_ref):
    w = lax.dot_general(w_ref[...], h_ref[...], (((1,), (1,)), ((), ())),
                        preferred_element_type=jnp.float32)
    s_ref[...] = jax.nn.sigmoid(w + b_ref[...])


_scores_call = pl.pallas_call(
    _scores_body,
    out_shape=jax.ShapeDtypeStruct((1, N), jnp.float32),
)


def _rank_body(s_col_ref, s_row_ref, idx_ref, s_ref):
    s_col = s_col_ref[...]
    CH = 512
    tri = (lax.broadcasted_iota(jnp.int32, (CH, CH), 1)
           < lax.broadcasted_iota(jnp.int32, (CH, CH), 0))
    rank = jnp.zeros((N, 1), jnp.float32)
    for c in range(N // CH):
        s_chunk = s_row_ref[:, c * CH:(c + 1) * CH]
        lo, hi = c * CH, (c + 1) * CH
        parts = []
        if lo > 0:
            parts.append(jnp.sum(
                jnp.where(s_chunk > s_col[:lo], 1.0, 0.0),
                axis=1, keepdims=True))
        sm = s_col[lo:hi]
        f = jnp.logical_or(s_chunk > sm,
                           jnp.logical_and(s_chunk == sm, tri))
        parts.append(jnp.sum(jnp.where(f, 1.0, 0.0), axis=1, keepdims=True))
        if hi < N:
            parts.append(jnp.sum(
                jnp.where(s_chunk >= s_col[hi:], 1.0, 0.0),
                axis=1, keepdims=True))
        part = parts[0] if len(parts) == 1 else jnp.concatenate(parts, axis=0)
        rank = rank + part
    i_colf = lax.broadcasted_iota(jnp.int32, (N, 1), 0).astype(jnp.float32)
    for pc in range(K // CH):
        p_iota = (lax.broadcasted_iota(jnp.int32, (N, CH), 1)
                  + pc * CH).astype(jnp.float32)
        m = rank == p_iota
        s_ref[:, pc * CH:(pc + 1) * CH] = jnp.where(
            m, 1.0, 0.0).astype(jnp.bfloat16)
        sel = jnp.sum(jnp.where(m, i_colf, 0.0), axis=0, keepdims=True)
        idx_ref[:, pc * CH:(pc + 1) * CH] = sel.astype(jnp.int32)


_rank_call = pl.pallas_call(
    _rank_body,
    out_shape=(jax.ShapeDtypeStruct((1, K), jnp.int32),
               jax.ShapeDtypeStruct((N, K), jnp.bfloat16)),
)


def _bits_body(idx_ref, g_ref, o_ref):
    o_ref[...] = (g_ref[...] != 0.0).astype(jnp.bfloat16)


_bits_call = pl.pallas_call(
    _bits_body,
    grid_spec=pltpu.PrefetchScalarGridSpec(
        num_scalar_prefetch=1,
        grid=(T_TC,),
        in_specs=[pl.BlockSpec((1, 1, N),
                               lambda i, idx_ref: (idx_ref[i], 0, 0))],
        out_specs=pl.BlockSpec((1, 1, N), lambda i, idx_ref: (i, 0, 0)),
    ),
    out_shape=jax.ShapeDtypeStruct((T_TC, 1, N), jnp.bfloat16),
)


def _mm_body(bits_ref, s_ref, o_ref):
    p = lax.dot_general(bits_ref[...], s_ref[...], (((1,), (0,)), ((), ())),
                        preferred_element_type=jnp.float32)
    cnt = jnp.sum(p, axis=1, keepdims=True)
    o_ref[...] = p / cnt


_mm_call = pl.pallas_call(
    _mm_body,
    grid=(T_TC // RB,),
    in_specs=[pl.BlockSpec((RB, N), lambda i: (i, 0)),
              pl.BlockSpec((N, K), lambda i: (0, 0))],
    out_specs=pl.BlockSpec((RB, K), lambda i: (i, 0)),
    out_shape=jax.ShapeDtypeStruct((T_TC, K), jnp.float32),
)


def _sc_pool_impl(g_hbm, h_hbm, idx_hbm, idxr_hbm, lut_hbm, gnew_hbm,
                  newh_hbm, idx_v, cidx_v, hrows_v, g0_v, g1_v, obuf_v, lut_v,
                  sem_h, sem_g0, sem_g1, sem_o):
    wid = lax.axis_index("s") * NC + lax.axis_index("c")
    hbase = wid * RPW_H
    gbase = T_TC + wid * RPW_G
    pltpu.sync_copy(idx_hbm, idx_v)
    pltpu.sync_copy(lut_hbm, lut_v)
    pltpu.sync_copy(idxr_hbm.at[pl.ds(wid * NCHUNK, NCHUNK)], cidx_v)
    cp_h = pltpu.async_copy(h_hbm.at[idx_v.at[pl.ds(hbase, RPW_H)]], hrows_v,
                            sem_h)
    gbufs = (g0_v, g1_v)
    sems = (sem_g0, sem_g1)
    splats = [jnp.full((L,), r, jnp.int32) for r in range(GCHUNK)]
    pltpu.async_copy(g_hbm.at[cidx_v.at[0]], g0_v, sem_g0)

    def outer(t, _):
        for b in range(2):
            c = t * 2 + b
            nxt = c + 1

            @pl.when(nxt < NCHUNK)
            def _():
                pltpu.async_copy(g_hbm.at[cidx_v.at[nxt]], gbufs[1 - b],
                                 sems[1 - b])

            pltpu.make_async_copy(g_hbm.at[pl.ds(0, GCHUNK)], gbufs[b],
                                  sems[b]).wait()
            gbuf = gbufs[b]
            zeros = jnp.zeros((L,), jnp.int32)

            @plsc.parallel_loop(0, K // L, unroll=UNROLL,
                                carry=(zeros,) * GCHUNK)
            def counts(k, accs):
                colv = idx_v[pl.ds(k * L, L)]
                return tuple(
                    accs[r] + plsc.all_reduce_population_count(
                        plsc.load_gather(gbuf, [splats[r], colv]) != 0.0)
                    for r in range(GCHUNK))

            invs = [plsc.load_gather(lut_v, [counts[r]])
                    for r in range(GCHUNK)]

            @pl.when(c > 0)
            def _():
                pltpu.make_async_copy(
                    gnew_hbm.at[pl.ds(0, GCHUNK)], obuf_v, sem_o).wait()

            @plsc.parallel_loop(0, K // L, unroll=UNROLL)
            def _(k):
                colv = idx_v[pl.ds(k * L, L)]
                for r in range(GCHUNK):
                    vals = plsc.load_gather(gbuf, [splats[r], colv])
                    obuf_v[r, pl.ds(k * L, L)] = jnp.where(
                        vals != 0.0, invs[r], 0.0)

            pltpu.async_copy(obuf_v,
                             gnew_hbm.at[pl.ds(gbase + c * GCHUNK, GCHUNK)],
                             sem_o)
        return 0

    lax.fori_loop(0, NCHUNK // 2, outer, 0)
    pltpu.make_async_copy(gnew_hbm.at[pl.ds(0, GCHUNK)], obuf_v, sem_o).wait()
    cp_h.wait()
    pltpu.sync_copy(hrows_v, newh_hbm.at[pl.ds(hbase, RPW_H)])


@functools.lru_cache(maxsize=1)
def _get_sc_pool():
    mesh = plsc.VectorSubcoreMesh(core_axis_name="c", subcore_axis_name="s",
                                  num_cores=NC, num_subcores=NS)
    return pl.kernel(
        _sc_pool_impl,
        out_type=(jax.ShapeDtypeStruct((K, K), jnp.float32),
                  jax.ShapeDtypeStruct((K, D), jnp.float32)),
        mesh=mesh,
        compiler_params=pltpu.CompilerParams(needs_layout_passes=False,
                                             use_tc_tiling_on_sc=False),
        scratch_types=[pltpu.VMEM((K,), jnp.int32),
                       pltpu.VMEM((NCHUNK, GCHUNK), jnp.int32),
                       pltpu.VMEM((RPW_H, D), jnp.float32),
                       pltpu.VMEM((GCHUNK, N), jnp.float32),
                       pltpu.VMEM((GCHUNK, N), jnp.float32),
                       pltpu.VMEM((GCHUNK, K), jnp.float32),
                       pltpu.VMEM((K + 1,), jnp.float32),
                       pltpu.SemaphoreType.DMA,
                       pltpu.SemaphoreType.DMA,
                       pltpu.SemaphoreType.DMA,
                       pltpu.SemaphoreType.DMA],
    )


def kernel(g, h, W, b):
    scores2d = _scores_call(h, W, b.reshape(1, 1))
    idx2d, s_onehot = _rank_call(scores2d.reshape(N, 1), scores2d)
    idx = idx2d.reshape(K)
    lut = 1.0 / jnp.arange(K + 1, dtype=jnp.float32)
    idx_sc = idx[T_TC:].reshape(SC_ROWS // GCHUNK, GCHUNK)
    gnew_sc, new_h = _get_sc_pool()(g, h, idx, idx_sc, lut)
    bits = _bits_call(idx, g.reshape(N, 1, N)).reshape(T_TC, N)
    gnew_top = _mm_call(bits, s_onehot)
    g_new = lax.dynamic_update_slice(gnew_sc, gnew_top, (0, 0))
    return (g_new, new_h, idx, scores2d.reshape(N))


# speedup vs baseline: 7.7010x; 7.7010x over previous
import functools

import jax
import jax.numpy as jnp
from jax import lax
from jax.experimental import pallas as pl
from jax.experimental.pallas import tpu as pltpu
from jax.experimental.pallas import tpu_sc as plsc

N = 4096
D = 512
K = 2048
NC, NS = 2, 16
NW = NC * NS
RPW = K // NW
GCHUNK = 8
L = 16


def _scores_body(h_ref, w_ref, b_ref, s_ref):
    w = lax.dot_general(w_ref[...], h_ref[...], (((1,), (1,)), ((), ())),
                        preferred_element_type=jnp.float32)
    s_ref[...] = jax.nn.sigmoid(w + b_ref[...])


_scores_call = pl.pallas_call(
    _scores_body,
    out_shape=jax.ShapeDtypeStruct((1, N), jnp.float32),
)


def _rank_body(s_col_ref, s_row_ref, idx_ref):
    s_col = s_col_ref[...]
    CH = 512
    tri = (lax.broadcasted_iota(jnp.int32, (CH, CH), 1)
           < lax.broadcasted_iota(jnp.int32, (CH, CH), 0))
    rank = jnp.zeros((N, 1), jnp.float32)
    for c in range(N // CH):
        s_chunk = s_row_ref[:, c * CH:(c + 1) * CH]
        lo, hi = c * CH, (c + 1) * CH
        parts = []
        if lo > 0:
            parts.append(jnp.sum(
                jnp.where(s_chunk > s_col[:lo], 1.0, 0.0),
                axis=1, keepdims=True))
        sm = s_col[lo:hi]
        f = jnp.logical_or(s_chunk > sm,
                           jnp.logical_and(s_chunk == sm, tri))
        parts.append(jnp.sum(jnp.where(f, 1.0, 0.0), axis=1, keepdims=True))
        if hi < N:
            parts.append(jnp.sum(
                jnp.where(s_chunk >= s_col[hi:], 1.0, 0.0),
                axis=1, keepdims=True))
        part = parts[0] if len(parts) == 1 else jnp.concatenate(parts, axis=0)
        rank = rank + part
    i_colf = lax.broadcasted_iota(jnp.int32, (N, 1), 0).astype(jnp.float32)
    for pc in range(K // CH):
        p_iota = (lax.broadcasted_iota(jnp.int32, (N, CH), 1)
                  + pc * CH).astype(jnp.float32)
        m = rank == p_iota
        sel = jnp.sum(jnp.where(m, i_colf, 0.0), axis=0, keepdims=True)
        idx_ref[:, pc * CH:(pc + 1) * CH] = sel.astype(jnp.int32)


_rank_call = pl.pallas_call(
    _rank_body,
    out_shape=jax.ShapeDtypeStruct((1, K), jnp.int32),
)


NCHUNK = RPW // GCHUNK
UNROLL = 2


def _sc_pool_impl(g_hbm, h_hbm, idx_hbm, idxr_hbm, lut_hbm, gnew_hbm,
                  newh_hbm, idx_v, cidx_v, hrows_v, g0_v, g1_v, obuf_v, lut_v,
                  sem_h, sem_g0, sem_g1, sem_o):
    wid = lax.axis_index("s") * NC + lax.axis_index("c")
    base = wid * RPW
    pltpu.sync_copy(idx_hbm, idx_v)
    pltpu.sync_copy(lut_hbm, lut_v)
    pltpu.sync_copy(idxr_hbm.at[pl.ds(wid * NCHUNK, NCHUNK)], cidx_v)
    cp_h = pltpu.async_copy(h_hbm.at[idx_v.at[pl.ds(base, RPW)]], hrows_v,
                            sem_h)
    gbufs = (g0_v, g1_v)
    sems = (sem_g0, sem_g1)
    splats = [jnp.full((L,), r, jnp.int32) for r in range(GCHUNK)]
    pltpu.async_copy(g_hbm.at[cidx_v.at[0]], g0_v, sem_g0)

    def outer(t, _):
        for b in range(2):
            c = t * 2 + b
            nxt = c + 1

            @pl.when(nxt < NCHUNK)
            def _():
                pltpu.async_copy(g_hbm.at[cidx_v.at[nxt]], gbufs[1 - b],
                                 sems[1 - b])

            pltpu.make_async_copy(g_hbm.at[pl.ds(0, GCHUNK)], gbufs[b],
                                  sems[b]).wait()
            gbuf = gbufs[b]
            zeros = jnp.zeros((L,), jnp.int32)

            @plsc.parallel_loop(0, K // L, unroll=UNROLL,
                                carry=(zeros,) * GCHUNK)
            def counts(k, accs):
                colv = idx_v[pl.ds(k * L, L)]
                return tuple(
                    accs[r] + plsc.all_reduce_population_count(
                        plsc.load_gather(gbuf, [splats[r], colv]) != 0.0)
                    for r in range(GCHUNK))

            invs = [plsc.load_gather(lut_v, [counts[r]])
                    for r in range(GCHUNK)]

            @pl.when(c > 0)
            def _():
                pltpu.make_async_copy(
                    gnew_hbm.at[pl.ds(0, GCHUNK)], obuf_v, sem_o).wait()

            @plsc.parallel_loop(0, K // L, unroll=UNROLL)
            def _(k):
                colv = idx_v[pl.ds(k * L, L)]
                for r in range(GCHUNK):
                    vals = plsc.load_gather(gbuf, [splats[r], colv])
                    obuf_v[r, pl.ds(k * L, L)] = jnp.where(
                        vals != 0.0, invs[r], 0.0)

            pltpu.async_copy(obuf_v,
                             gnew_hbm.at[pl.ds(base + c * GCHUNK, GCHUNK)],
                             sem_o)
        return 0

    lax.fori_loop(0, NCHUNK // 2, outer, 0)
    pltpu.make_async_copy(gnew_hbm.at[pl.ds(0, GCHUNK)], obuf_v, sem_o).wait()
    cp_h.wait()
    pltpu.sync_copy(hrows_v, newh_hbm.at[pl.ds(base, RPW)])


@functools.lru_cache(maxsize=1)
def _get_sc_pool():
    mesh = plsc.VectorSubcoreMesh(core_axis_name="c", subcore_axis_name="s",
                                  num_cores=NC, num_subcores=NS)
    return pl.kernel(
        _sc_pool_impl,
        out_type=(jax.ShapeDtypeStruct((K, K), jnp.float32),
                  jax.ShapeDtypeStruct((K, D), jnp.float32)),
        mesh=mesh,
        compiler_params=pltpu.CompilerParams(needs_layout_passes=False),
        scratch_types=[pltpu.VMEM((K,), jnp.int32),
                       pltpu.VMEM((NCHUNK, GCHUNK), jnp.int32),
                       pltpu.VMEM((RPW, D), jnp.float32),
                       pltpu.VMEM((GCHUNK, N), jnp.float32),
                       pltpu.VMEM((GCHUNK, N), jnp.float32),
                       pltpu.VMEM((GCHUNK, K), jnp.float32),
                       pltpu.VMEM((K + 1,), jnp.float32),
                       pltpu.SemaphoreType.DMA,
                       pltpu.SemaphoreType.DMA,
                       pltpu.SemaphoreType.DMA,
                       pltpu.SemaphoreType.DMA],
    )


def kernel(g, h, W, b):
    scores2d = _scores_call(h, W, b.reshape(1, 1))
    idx2d = _rank_call(scores2d.reshape(N, 1), scores2d)
    idx = idx2d.reshape(K)
    lut = 1.0 / jnp.arange(K + 1, dtype=jnp.float32)
    g_new, new_h = _get_sc_pool()(g, h, idx, idx.reshape(K // GCHUNK, GCHUNK),
                                  lut)
    return (g_new, new_h, idx, scores2d.reshape(N))
